# baseline (speedup 1.0000x reference)
import functools

import numpy as np
import jax
import jax.numpy as jnp
from jax.experimental import pallas as pl
from jax.experimental.pallas import tpu as pltpu

_BEFORE = 3
_AFTER = 3
_W = _BEFORE + 1 + _AFTER


def _unfold_kernel(x_hbm, o_ref, xpad, sem, *, tb, t_total, b, d):
    i = pl.program_id(0)
    nblk = pl.num_programs(0)
    nq = (d * _W) // 128

    @pl.when((i == 0) | (i == nblk // 2))
    def _fill_scratch():
        cp = pltpu.make_async_copy(x_hbm, xpad.at[pl.ds(_BEFORE, t_total)], sem)
        cp.start()
        cp.wait()
        for r in range(_BEFORE):
            xpad[pl.ds(r, 1)] = xpad[pl.ds(_BEFORE, 1)]
        for r in range(_AFTER):
            xpad[pl.ds(t_total + _BEFORE + r, 1)] = \
                xpad[pl.ds(t_total + _BEFORE - 1, 1)]

    base = i * tb
    tg = 32
    nrows = tg + _W - 1
    lanes = jax.lax.broadcasted_iota(jnp.int32, (1, 1, 128), 2)
    for q in range(nq):
        c = lanes + (128 * q)
        cdiv7 = (c * 9363) >> 16
        cmod7 = c - 7 * cdiv7
        h = q // _W
        idx = jnp.broadcast_to(cdiv7 - 128 * h, (nrows, b, 128))
        masks = [cmod7 == w for w in range(_W)]
        for t8 in range(0, tb, tg):
            rows = xpad[pl.ds(base + t8, nrows), :, 128 * h:128 * (h + 1)]
            g = jnp.take_along_axis(rows, idx, axis=2)
            acc = g[0:tg]
            for w in range(1, _W):
                acc = jnp.where(masks[w], g[w:w + tg], acc)
            o_ref[pl.ds(t8, tg), :, 128 * q:128 * (q + 1)] = acc


def kernel(padded_data):
    t_total, b, d = padded_data.shape
    tb = 128
    grid = (t_total // tb,)
    body = functools.partial(
        _unfold_kernel, tb=tb, t_total=t_total, b=b, d=d)
    out = pl.pallas_call(
        body,
        grid=grid,
        in_specs=[pl.BlockSpec(memory_space=pl.ANY)],
        out_specs=pl.BlockSpec((tb, b, d * _W), lambda i: (i, 0, 0)),
        out_shape=jax.ShapeDtypeStruct((t_total, b, d * _W), padded_data.dtype),
        scratch_shapes=[
            pltpu.VMEM((t_total + _W - 1, b, d), padded_data.dtype),
            pltpu.SemaphoreType.DMA,
        ],
        compiler_params=pltpu.CompilerParams(
            dimension_semantics=("parallel",),
        ),
    )(padded_data)
    return out


# speedup vs baseline: 1.1863x; 1.1863x over previous
import functools

import jax
import jax.numpy as jnp
from jax.experimental import pallas as pl
from jax.experimental.pallas import tpu as pltpu

_BEFORE = 3
_AFTER = 3
_W = _BEFORE + 1 + _AFTER


def _start_block_dma(x_hbm, xpad, sems, j, slot, *, tb, t_total):
    nblk = t_total // tb

    @pl.when(j == 0)
    def _():
        pltpu.make_async_copy(
            x_hbm.at[pl.ds(0, tb + _AFTER)],
            xpad.at[slot, pl.ds(_BEFORE, tb + _AFTER)],
            sems.at[slot],
        ).start()

    @pl.when((j > 0) & (j < nblk - 1))
    def _():
        pltpu.make_async_copy(
            x_hbm.at[pl.ds(j * tb - _BEFORE, tb + _BEFORE + _AFTER)],
            xpad.at[slot],
            sems.at[slot],
        ).start()

    @pl.when(j == nblk - 1)
    def _():
        pltpu.make_async_copy(
            x_hbm.at[pl.ds(t_total - tb - _BEFORE, tb + _BEFORE)],
            xpad.at[slot, pl.ds(0, tb + _BEFORE)],
            sems.at[slot],
        ).start()


def _unfold_kernel(x_hbm, o_ref, xpad, sems, *, tb, t_total, b, d):
    i = pl.program_id(0)
    nblk = pl.num_programs(0)
    nq = (d * _W) // 128
    start = functools.partial(_start_block_dma, x_hbm, xpad, sems,
                              tb=tb, t_total=t_total)

    @pl.when(i == 0)
    def _prologue():
        start(0, 0)

    @pl.when(i < nblk - 1)
    def _prefetch_next():
        start(i + 1, (i + 1) % 2)

    slot = i % 2

    @pl.when(i == 0)
    def _wait_first():
        pltpu.make_async_copy(
            x_hbm.at[pl.ds(0, tb + _AFTER)],
            xpad.at[slot, pl.ds(_BEFORE, tb + _AFTER)],
            sems.at[slot],
        ).wait()

    @pl.when((i > 0) & (i < nblk - 1))
    def _wait_mid():
        pltpu.make_async_copy(
            x_hbm.at[pl.ds(i * tb - _BEFORE, tb + _BEFORE + _AFTER)],
            xpad.at[slot],
            sems.at[slot],
        ).wait()

    @pl.when(i == nblk - 1)
    def _wait_last():
        pltpu.make_async_copy(
            x_hbm.at[pl.ds(t_total - tb - _BEFORE, tb + _BEFORE)],
            xpad.at[slot, pl.ds(0, tb + _BEFORE)],
            sems.at[slot],
        ).wait()

    @pl.when(i == 0)
    def _dup_head():
        for r in range(_BEFORE):
            xpad[slot, pl.ds(r, 1)] = xpad[slot, pl.ds(_BEFORE, 1)]

    @pl.when(i == nblk - 1)
    def _dup_tail():
        for r in range(_AFTER):
            xpad[slot, pl.ds(tb + _BEFORE + r, 1)] = \
                xpad[slot, pl.ds(tb + _BEFORE - 1, 1)]

    tg = 32
    nrows = tg + _W - 1
    lanes = jax.lax.broadcasted_iota(jnp.int32, (1, 1, 128), 2)
    for q in range(nq):
        c = lanes + (128 * q)
        cdiv7 = (c * 9363) >> 16
        cmod7 = c - 7 * cdiv7
        h = q // _W
        idx = jnp.broadcast_to(cdiv7 - 128 * h, (nrows, b, 128))
        masks = [cmod7 == w for w in range(_W)]
        for t8 in range(0, tb, tg):
            rows = xpad[slot, pl.ds(t8, nrows), :, 128 * h:128 * (h + 1)]
            g = jnp.take_along_axis(rows, idx, axis=2)
            acc = g[0:tg]
            for w in range(1, _W):
                acc = jnp.where(masks[w], g[w:w + tg], acc)
            o_ref[pl.ds(t8, tg), :, 128 * q:128 * (q + 1)] = acc


def kernel(padded_data):
    t_total, b, d = padded_data.shape
    tb = 128
    grid = (t_total // tb,)
    body = functools.partial(
        _unfold_kernel, tb=tb, t_total=t_total, b=b, d=d)
    out = pl.pallas_call(
        body,
        grid=grid,
        in_specs=[pl.BlockSpec(memory_space=pl.ANY)],
        out_specs=pl.BlockSpec((tb, b, d * _W), lambda i: (i, 0, 0)),
        out_shape=jax.ShapeDtypeStruct((t_total, b, d * _W), padded_data.dtype),
        scratch_shapes=[
            pltpu.VMEM((2, tb + _W - 1, b, d), padded_data.dtype),
            pltpu.SemaphoreType.DMA((2,)),
        ],
        compiler_params=pltpu.CompilerParams(
            dimension_semantics=("arbitrary",),
        ),
    )(padded_data)
    return out


# speedup vs baseline: 1.2282x; 1.0353x over previous
import functools

import jax
import jax.numpy as jnp
from jax.experimental import pallas as pl
from jax.experimental.pallas import tpu as pltpu

_BEFORE = 3
_AFTER = 3
_W = _BEFORE + 1 + _AFTER


def _start_block_dma(x_hbm, xpad, sems, j, slot, *, tb, t_total):
    nblk = t_total // tb

    @pl.when(j == 0)
    def _():
        pltpu.make_async_copy(
            x_hbm.at[pl.ds(0, tb + _AFTER)],
            xpad.at[slot, pl.ds(_BEFORE, tb + _AFTER)],
            sems.at[slot],
        ).start()

    @pl.when((j > 0) & (j < nblk - 1))
    def _():
        pltpu.make_async_copy(
            x_hbm.at[pl.ds(j * tb - _BEFORE, tb + _BEFORE + _AFTER)],
            xpad.at[slot],
            sems.at[slot],
        ).start()

    @pl.when(j == nblk - 1)
    def _():
        pltpu.make_async_copy(
            x_hbm.at[pl.ds(t_total - tb - _BEFORE, tb + _BEFORE)],
            xpad.at[slot, pl.ds(0, tb + _BEFORE)],
            sems.at[slot],
        ).start()


def _unfold_kernel(x_hbm, o_ref, xpad, sems, *, tb, t_total, b, d):
    i = pl.program_id(0)
    nblk = pl.num_programs(0)
    nq = (d * _W) // 128
    start = functools.partial(_start_block_dma, x_hbm, xpad, sems,
                              tb=tb, t_total=t_total)

    @pl.when(i == 0)
    def _prologue():
        start(0, 0)

    @pl.when(i < nblk - 1)
    def _prefetch_next():
        start(i + 1, (i + 1) % 2)

    slot = i % 2

    @pl.when(i == 0)
    def _wait_first():
        pltpu.make_async_copy(
            x_hbm.at[pl.ds(0, tb + _AFTER)],
            xpad.at[slot, pl.ds(_BEFORE, tb + _AFTER)],
            sems.at[slot],
        ).wait()

    @pl.when((i > 0) & (i < nblk - 1))
    def _wait_mid():
        pltpu.make_async_copy(
            x_hbm.at[pl.ds(i * tb - _BEFORE, tb + _BEFORE + _AFTER)],
            xpad.at[slot],
            sems.at[slot],
        ).wait()

    @pl.when(i == nblk - 1)
    def _wait_last():
        pltpu.make_async_copy(
            x_hbm.at[pl.ds(t_total - tb - _BEFORE, tb + _BEFORE)],
            xpad.at[slot, pl.ds(0, tb + _BEFORE)],
            sems.at[slot],
        ).wait()

    @pl.when(i == 0)
    def _dup_head():
        for r in range(_BEFORE):
            xpad[slot, pl.ds(r, 1)] = xpad[slot, pl.ds(_BEFORE, 1)]

    @pl.when(i == nblk - 1)
    def _dup_tail():
        for r in range(_AFTER):
            xpad[slot, pl.ds(tb + _BEFORE + r, 1)] = \
                xpad[slot, pl.ds(tb + _BEFORE - 1, 1)]

    tg = 128
    nrows = tg + _W - 1
    lanes = jax.lax.broadcasted_iota(jnp.int32, (1, 1, 128), 2)
    for q in range(nq):
        c = lanes + (128 * q)
        cdiv7 = (c * 9363) >> 16
        cmod7 = c - 7 * cdiv7
        h = q // _W
        idx = jnp.broadcast_to(cdiv7 - 128 * h, (nrows, b, 128))
        masks = [cmod7 == w for w in range(_W)]
        for t8 in range(0, tb, tg):
            rows = xpad[slot, pl.ds(t8, nrows), :, 128 * h:128 * (h + 1)]
            g = jnp.take_along_axis(rows, idx, axis=2)
            acc = g[0:tg]
            for w in range(1, _W):
                acc = jnp.where(masks[w], g[w:w + tg], acc)
            o_ref[pl.ds(t8, tg), :, 128 * q:128 * (q + 1)] = acc


def kernel(padded_data):
    t_total, b, d = padded_data.shape
    tb = 128
    grid = (t_total // tb,)
    body = functools.partial(
        _unfold_kernel, tb=tb, t_total=t_total, b=b, d=d)
    out = pl.pallas_call(
        body,
        grid=grid,
        in_specs=[pl.BlockSpec(memory_space=pl.ANY)],
        out_specs=pl.BlockSpec((tb, b, d * _W), lambda i: (i, 0, 0)),
        out_shape=jax.ShapeDtypeStruct((t_total, b, d * _W), padded_data.dtype),
        scratch_shapes=[
            pltpu.VMEM((2, tb + _W - 1, b, d), padded_data.dtype),
            pltpu.SemaphoreType.DMA((2,)),
        ],
        compiler_params=pltpu.CompilerParams(
            dimension_semantics=("arbitrary",),
        ),
    )(padded_data)
    return out


# speedup vs baseline: 1.2735x; 1.0369x over previous
import functools

import jax
import jax.numpy as jnp
from jax.experimental import pallas as pl
from jax.experimental.pallas import tpu as pltpu

_BEFORE = 3
_AFTER = 3
_W = _BEFORE + 1 + _AFTER


def _start_block_dma(x_hbm, xpad, sems, j, slot, *, tb, t_total):
    nblk = t_total // tb

    @pl.when(j == 0)
    def _():
        pltpu.make_async_copy(
            x_hbm.at[pl.ds(0, tb + _AFTER)],
            xpad.at[slot, pl.ds(_BEFORE, tb + _AFTER)],
            sems.at[slot],
        ).start()

    @pl.when((j > 0) & (j < nblk - 1))
    def _():
        pltpu.make_async_copy(
            x_hbm.at[pl.ds(j * tb - _BEFORE, tb + _BEFORE + _AFTER)],
            xpad.at[slot],
            sems.at[slot],
        ).start()

    @pl.when(j == nblk - 1)
    def _():
        pltpu.make_async_copy(
            x_hbm.at[pl.ds(t_total - tb - _BEFORE, tb + _BEFORE)],
            xpad.at[slot, pl.ds(0, tb + _BEFORE)],
            sems.at[slot],
        ).start()


def _unfold_kernel(x_hbm, o_ref, xpad, sems, *, tb, t_total, b, d):
    i = pl.program_id(0)
    nblk = pl.num_programs(0)
    nq = (d * _W) // 128
    start = functools.partial(_start_block_dma, x_hbm, xpad, sems,
                              tb=tb, t_total=t_total)

    @pl.when(i == 0)
    def _prologue():
        start(0, 0)

    @pl.when(i < nblk - 1)
    def _prefetch_next():
        start(i + 1, (i + 1) % 2)

    slot = i % 2

    @pl.when(i == 0)
    def _wait_first():
        pltpu.make_async_copy(
            x_hbm.at[pl.ds(0, tb + _AFTER)],
            xpad.at[slot, pl.ds(_BEFORE, tb + _AFTER)],
            sems.at[slot],
        ).wait()

    @pl.when((i > 0) & (i < nblk - 1))
    def _wait_mid():
        pltpu.make_async_copy(
            x_hbm.at[pl.ds(i * tb - _BEFORE, tb + _BEFORE + _AFTER)],
            xpad.at[slot],
            sems.at[slot],
        ).wait()

    @pl.when(i == nblk - 1)
    def _wait_last():
        pltpu.make_async_copy(
            x_hbm.at[pl.ds(t_total - tb - _BEFORE, tb + _BEFORE)],
            xpad.at[slot, pl.ds(0, tb + _BEFORE)],
            sems.at[slot],
        ).wait()

    @pl.when(i == 0)
    def _dup_head():
        for r in range(_BEFORE):
            xpad[slot, pl.ds(r, 1)] = xpad[slot, pl.ds(_BEFORE, 1)]

    @pl.when(i == nblk - 1)
    def _dup_tail():
        for r in range(_AFTER):
            xpad[slot, pl.ds(tb + _BEFORE + r, 1)] = \
                xpad[slot, pl.ds(tb + _BEFORE - 1, 1)]

    tg = 128
    nrows = tg + _W - 1
    lanes = jax.lax.broadcasted_iota(jnp.int32, (1, 1, 128), 2)
    for q in range(nq):
        c = lanes + (128 * q)
        cdiv7 = (c * 9363) >> 16
        cmod7 = c - 7 * cdiv7
        h = q // _W
        idx = jnp.broadcast_to(cdiv7 - 128 * h, (nrows, b, 128))
        masks = [cmod7 == w for w in range(_W)]
        for t8 in range(0, tb, tg):
            rows = xpad[slot, pl.ds(t8, nrows), :, 128 * h:128 * (h + 1)]
            g = jnp.take_along_axis(rows, idx, axis=2)
            acc = g[0:tg]
            for w in range(1, _W):
                acc = jnp.where(masks[w], g[w:w + tg], acc)
            o_ref[pl.ds(t8, tg), :, 128 * q:128 * (q + 1)] = acc


def kernel(padded_data):
    t_total, b, d = padded_data.shape
    tb = 256
    grid = (t_total // tb,)
    body = functools.partial(
        _unfold_kernel, tb=tb, t_total=t_total, b=b, d=d)
    out = pl.pallas_call(
        body,
        grid=grid,
        in_specs=[pl.BlockSpec(memory_space=pl.ANY)],
        out_specs=pl.BlockSpec((tb, b, d * _W), lambda i: (i, 0, 0)),
        out_shape=jax.ShapeDtypeStruct((t_total, b, d * _W), padded_data.dtype),
        scratch_shapes=[
            pltpu.VMEM((2, tb + _W - 1, b, d), padded_data.dtype),
            pltpu.SemaphoreType.DMA((2,)),
        ],
        compiler_params=pltpu.CompilerParams(
            dimension_semantics=("arbitrary",),
        ),
    )(padded_data)
    return out


# speedup vs baseline: 1.2781x; 1.0036x over previous
import functools

import jax
import jax.numpy as jnp
from jax.experimental import pallas as pl
from jax.experimental.pallas import tpu as pltpu

_BEFORE = 3
_AFTER = 3
_W = _BEFORE + 1 + _AFTER


def _start_block_dma(x_hbm, xpad, sems, j, slot, *, tb, t_total):
    nblk = t_total // tb

    @pl.when(j == 0)
    def _():
        pltpu.make_async_copy(
            x_hbm.at[pl.ds(0, tb + _AFTER)],
            xpad.at[slot, pl.ds(_BEFORE, tb + _AFTER)],
            sems.at[slot],
        ).start()

    @pl.when((j > 0) & (j < nblk - 1))
    def _():
        pltpu.make_async_copy(
            x_hbm.at[pl.ds(j * tb - _BEFORE, tb + _BEFORE + _AFTER)],
            xpad.at[slot],
            sems.at[slot],
        ).start()

    @pl.when(j == nblk - 1)
    def _():
        pltpu.make_async_copy(
            x_hbm.at[pl.ds(t_total - tb - _BEFORE, tb + _BEFORE)],
            xpad.at[slot, pl.ds(0, tb + _BEFORE)],
            sems.at[slot],
        ).start()


def _unfold_kernel(x_hbm, o_ref, xpad, sems, *, tb, t_total, b, d):
    i = pl.program_id(0)
    nblk = pl.num_programs(0)
    nq = (d * _W) // 128
    start = functools.partial(_start_block_dma, x_hbm, xpad, sems,
                              tb=tb, t_total=t_total)

    @pl.when(i == 0)
    def _prologue():
        start(0, 0)

    @pl.when(i < nblk - 1)
    def _prefetch_next():
        start(i + 1, (i + 1) % 2)

    slot = i % 2

    @pl.when(i == 0)
    def _wait_first():
        pltpu.make_async_copy(
            x_hbm.at[pl.ds(0, tb + _AFTER)],
            xpad.at[slot, pl.ds(_BEFORE, tb + _AFTER)],
            sems.at[slot],
        ).wait()

    @pl.when((i > 0) & (i < nblk - 1))
    def _wait_mid():
        pltpu.make_async_copy(
            x_hbm.at[pl.ds(i * tb - _BEFORE, tb + _BEFORE + _AFTER)],
            xpad.at[slot],
            sems.at[slot],
        ).wait()

    @pl.when(i == nblk - 1)
    def _wait_last():
        pltpu.make_async_copy(
            x_hbm.at[pl.ds(t_total - tb - _BEFORE, tb + _BEFORE)],
            xpad.at[slot, pl.ds(0, tb + _BEFORE)],
            sems.at[slot],
        ).wait()

    @pl.when(i == 0)
    def _dup_head():
        for r in range(_BEFORE):
            xpad[slot, pl.ds(r, 1)] = xpad[slot, pl.ds(_BEFORE, 1)]

    @pl.when(i == nblk - 1)
    def _dup_tail():
        for r in range(_AFTER):
            xpad[slot, pl.ds(tb + _BEFORE + r, 1)] = \
                xpad[slot, pl.ds(tb + _BEFORE - 1, 1)]

    tg = 256
    nrows = tg + _W - 1
    lanes = jax.lax.broadcasted_iota(jnp.int32, (1, 1, 128), 2)
    for q in range(nq):
        c = lanes + (128 * q)
        cdiv7 = (c * 9363) >> 16
        cmod7 = c - 7 * cdiv7
        h = q // _W
        idx = jnp.broadcast_to(cdiv7 - 128 * h, (nrows, b, 128))
        masks = [cmod7 == w for w in range(_W)]
        for t8 in range(0, tb, tg):
            rows = xpad[slot, pl.ds(t8, nrows), :, 128 * h:128 * (h + 1)]
            g = jnp.take_along_axis(rows, idx, axis=2)
            acc = g[0:tg]
            for w in range(1, _W):
                acc = jnp.where(masks[w], g[w:w + tg], acc)
            o_ref[pl.ds(t8, tg), :, 128 * q:128 * (q + 1)] = acc


def kernel(padded_data):
    t_total, b, d = padded_data.shape
    tb = 256
    grid = (t_total // tb,)
    body = functools.partial(
        _unfold_kernel, tb=tb, t_total=t_total, b=b, d=d)
    out = pl.pallas_call(
        body,
        grid=grid,
        in_specs=[pl.BlockSpec(memory_space=pl.ANY)],
        out_specs=pl.BlockSpec((tb, b, d * _W), lambda i: (i, 0, 0)),
        out_shape=jax.ShapeDtypeStruct((t_total, b, d * _W), padded_data.dtype),
        scratch_shapes=[
            pltpu.VMEM((2, tb + _W - 1, b, d), padded_data.dtype),
            pltpu.SemaphoreType.DMA((2,)),
        ],
        compiler_params=pltpu.CompilerParams(
            dimension_semantics=("arbitrary",),
        ),
    )(padded_data)
    return out
